# baseline (speedup 1.0000x reference)
import jax
import jax.numpy as jnp
from jax import lax
from jax.experimental import pallas as pl
from jax.experimental.pallas import tpu as pltpu
from jax.experimental.pallas import tpu_sc as plsc

N_POINTS = 1048576
D = 16
NBLK = N_POINTS // 128
TABLE_MASK = (1 << 20) - 1
RESOLUTION = 512.0
NC, NS = 2, 16
NW = NC * NS
BPW = NBLK // NW
NSB = BPW // 2
GROUP = 8
NGRP = BPW // GROUP

P1 = 2654435761
P2 = 805459861


def _wid():
    return lax.axis_index("s") * NC + lax.axis_index("c")


def _detile_body(ta_hbm, rm_hbm, b0, b1, rm0, rm1, sem, osem):
    wid = _wid()
    iota = lax.iota(jnp.int32, 16)
    bufs = (b0, b1)
    rms = (rm0, rm1)
    base = wid * NSB

    def start_in(sb, buf):
        for g in range(2):
            pltpu.async_copy(
                ta_hbm.at[g, pl.ds(2 * (base + sb), 2)], buf.at[g], sem)

    def wait_in(buf):
        for g in range(2):
            pltpu.make_async_copy(ta_hbm.at[0, pl.ds(0, 2)], buf.at[g], sem).wait()

    def wait_out(rm_v):
        pltpu.make_async_copy(rm_v.at[:, pl.ds(0, D)],
                              rm_hbm.at[pl.ds(0, 256)], osem).wait()

    start_in(0, bufs[0])

    def halfiter(j, sb, p):
        start_in(jnp.minimum(sb + 1, NSB - 1), bufs[1 - p])
        wait_in(bufs[p])

        @pl.when(j >= 1)
        def _():
            wait_out(rms[p])

        buf = bufs[p]
        rm_v = rms[p]

        @plsc.parallel_loop(0, 256, unroll=4)
        def _(q):
            blk = lax.shift_right_logical(q, 7)
            g = lax.bitwise_and(lax.shift_right_logical(q, 6), 1)
            c = lax.bitwise_and(lax.shift_right_logical(q, 3), 7)
            p0 = lax.bitwise_and(q, 7)
            vals = buf[g, blk, c, pl.ds(p0 * 16, 16)]
            rowv = iota + (blk * 128 + p0 * 16)
            colv = iota * 0 + (g * 8 + c)
            plsc.store_scatter(rm_v, [rowv, colv], vals)

        pltpu.async_copy(
            rm_v.at[:, pl.ds(0, D)],
            rm_hbm.at[pl.ds((base + sb) * 256, 256)], osem)

    def body(j, carry):
        halfiter(j, 2 * j, 0)
        halfiter(j, 2 * j + 1, 1)
        return carry

    lax.fori_loop(0, NSB // 2, body, 0)
    wait_out(rms[0])
    wait_out(rms[1])
    wait_in(bufs[0])


def _quant(v):
    t = (v + 0.5) * RESOLUTION
    t = jnp.minimum(jnp.maximum(t, 0.0), RESOLUTION - 1.0)
    return t.astype(jnp.int32).astype(jnp.uint32)


def _lookup_body(x_hbm, y_hbm, z_hbm, rm_hbm, out_hbm,
                 xv, yv, zv, idx_v, rows_v, ob0, ob1, sem, gsem, osem):
    wid = _wid()
    iota = lax.iota(jnp.int32, 16)
    obufs = (ob0, ob1)
    pbase0 = wid * (BPW * 128)
    gv = lax.shift_right_logical(iota, 3)
    cv = lax.bitwise_and(iota, 7)

    def start_xyz(g, p):
        for h, v in ((x_hbm, xv), (y_hbm, yv), (z_hbm, zv)):
            pltpu.async_copy(h.at[pl.ds(pbase0 + g * (GROUP * 128),
                                        GROUP * 128)], v.at[p], sem)

    def wait_xyz(p):
        for h, v in ((x_hbm, xv), (y_hbm, yv), (z_hbm, zv)):
            pltpu.make_async_copy(h.at[pl.ds(0, GROUP * 128)], v.at[p], sem).wait()

    def drain_out(ob):
        for k in range(GROUP):
            for g in range(2):
                pltpu.make_async_copy(
                    ob.at[k, g, :, pl.ds(0, 128)], out_hbm.at[0, 0], osem).wait()

    start_xyz(0, 0)

    def halfiter(i, grp, p):
        start_xyz(jnp.minimum(grp + 1, NGRP - 1), 1 - p)
        wait_xyz(p)

        @plsc.parallel_loop(0, GROUP * 128, step=16, unroll=2)
        def _(s):
            sl = pl.ds(s, 16)
            h = (_quant(xv[p, sl]) ^ (_quant(yv[p, sl]) * jnp.uint32(P1))
                 ^ (_quant(zv[p, sl]) * jnp.uint32(P2)))
            idx_v[lax.shift_right_logical(s, 7),
                  pl.ds(lax.bitwise_and(s, 127), 16)] = (
                h & jnp.uint32(TABLE_MASK)).astype(jnp.int32)

        for k in range(GROUP):
            pltpu.async_copy(rm_hbm.at[idx_v.at[k]],
                             rows_v.at[pl.ds(k * 128, 128)], gsem)

        @pl.when(i >= 1)
        def _():
            drain_out(obufs[p])

        ob = obufs[p]
        bbase = wid * BPW + grp * GROUP
        for k in range(GROUP):
            pltpu.make_async_copy(rm_hbm.at[idx_v.at[k]],
                                  rows_v.at[pl.ds(k * 128, 128)], gsem).wait()
            kvec = iota * 0 + k

            @plsc.parallel_loop(0, 128, unroll=4)
            def _(p0):
                vals = rows_v[k * 128 + p0, :]
                plsc.store_scatter(ob, [kvec, gv, cv, iota * 0 + p0], vals)

            for g in range(2):
                pltpu.async_copy(ob.at[k, g, :, pl.ds(0, 128)],
                                 out_hbm.at[g, bbase + k], osem)

    def body(i, carry):
        halfiter(i, 2 * i, 0)
        halfiter(i, 2 * i + 1, 1)
        return carry

    lax.fori_loop(0, NGRP // 2, body, 0)
    drain_out(obufs[0])
    drain_out(obufs[1])
    wait_xyz(0)


_mesh = plsc.VectorSubcoreMesh(
    core_axis_name="c", subcore_axis_name="s", num_cores=NC, num_subcores=NS
)
_params = pltpu.CompilerParams(use_tc_tiling_on_sc=False,
                               needs_layout_passes=False)

_detile = pl.kernel(
    _detile_body,
    out_type=jax.ShapeDtypeStruct((N_POINTS, D), jnp.float32),
    mesh=_mesh,
    scratch_types=[
        pltpu.VMEM((2, 2, 8, 128), jnp.float32),
        pltpu.VMEM((2, 2, 8, 128), jnp.float32),
        pltpu.VMEM((256, D + 1), jnp.float32),
        pltpu.VMEM((256, D + 1), jnp.float32),
        pltpu.SemaphoreType.DMA,
        pltpu.SemaphoreType.DMA,
    ],
    compiler_params=_params,
)

_lookup = pl.kernel(
    _lookup_body,
    out_type=jax.ShapeDtypeStruct((2, NBLK, 8, 128), jnp.float32),
    mesh=_mesh,
    scratch_types=[
        pltpu.VMEM((2, GROUP * 128), jnp.float32),
        pltpu.VMEM((2, GROUP * 128), jnp.float32),
        pltpu.VMEM((2, GROUP * 128), jnp.float32),
        pltpu.VMEM((GROUP, 128), jnp.int32),
        pltpu.VMEM((GROUP * 128, D), jnp.float32),
        pltpu.VMEM((GROUP, 2, 8, 129), jnp.float32),
        pltpu.VMEM((GROUP, 2, 8, 129), jnp.float32),
        pltpu.SemaphoreType.DMA,
        pltpu.SemaphoreType.DMA,
        pltpu.SemaphoreType.DMA,
    ],
    compiler_params=_params,
)


@jax.jit
def kernel(xyz, table):
    table_a = table.reshape(NBLK, 128, 2, 8).transpose(2, 0, 3, 1)
    table_rm = _detile(table_a)
    out_a = _lookup(xyz[:, 0], xyz[:, 1], xyz[:, 2], table_rm)
    return out_a.transpose(1, 3, 0, 2).reshape(N_POINTS, D)


# speedup vs baseline: 1.3439x; 1.3439x over previous
import jax
import jax.numpy as jnp
from jax import lax
from jax.experimental import pallas as pl
from jax.experimental.pallas import tpu as pltpu
from jax.experimental.pallas import tpu_sc as plsc

N_POINTS = 1048576
D = 16
NBLK = N_POINTS // 128
TABLE_MASK = (1 << 20) - 1
RESOLUTION = 512.0
NC, NS = 2, 16
NW = NC * NS
BPW = NBLK // NW
NSB = BPW // 2
GROUP = 8
NGRP = BPW // GROUP

P1 = 2654435761
P2 = 805459861


def _wid():
    return lax.axis_index("s") * NC + lax.axis_index("c")


def _detile_body(ta_hbm, rm_hbm, b0, b1, rm0, rm1, sem, osem):
    wid = _wid()
    iota = lax.iota(jnp.int32, 16)
    bufs = (b0, b1)
    rms = (rm0, rm1)
    base = wid * NSB

    def start_in(sb, buf):
        for g in range(2):
            pltpu.async_copy(
                ta_hbm.at[g, pl.ds(2 * (base + sb), 2)], buf.at[g], sem)

    def wait_in(buf):
        for g in range(2):
            pltpu.make_async_copy(ta_hbm.at[0, pl.ds(0, 2)], buf.at[g], sem).wait()

    def wait_out(rm_v):
        pltpu.make_async_copy(rm_v, rm_hbm.at[pl.ds(0, 256)], osem).wait()

    start_in(0, bufs[0])

    def halfiter(j, sb, p):
        start_in(jnp.minimum(sb + 1, NSB - 1), bufs[1 - p])
        wait_in(bufs[p])

        @pl.when(j >= 1)
        def _():
            wait_out(rms[p])

        buf = bufs[p]
        rm_v = rms[p]

        @plsc.parallel_loop(0, 256, unroll=4)
        def _(q):
            blk = lax.shift_right_logical(q, 7)
            g = lax.bitwise_and(lax.shift_right_logical(q, 6), 1)
            c = lax.bitwise_and(lax.shift_right_logical(q, 3), 7)
            p0 = lax.bitwise_and(q, 7)
            vals = buf[g, blk, c, pl.ds(p0 * 16, 16)]
            rowv = iota + (blk * 128 + p0 * 16)
            colv = iota * 0 + (g * 8 + c)
            plsc.store_scatter(rm_v, [rowv, colv], vals)

        pltpu.async_copy(
            rm_v, rm_hbm.at[pl.ds((base + sb) * 256, 256)], osem)

    def body(j, carry):
        halfiter(j, 2 * j, 0)
        halfiter(j, 2 * j + 1, 1)
        return carry

    lax.fori_loop(0, NSB // 2, body, 0)
    wait_out(rms[0])
    wait_out(rms[1])
    wait_in(bufs[0])


def _quant(v):
    t = (v + 0.5) * RESOLUTION
    t = jnp.minimum(jnp.maximum(t, 0.0), RESOLUTION - 1.0)
    return t.astype(jnp.int32).astype(jnp.uint32)


def _lookup_body(x_hbm, y_hbm, z_hbm, rm_hbm, out_hbm,
                 xv, yv, zv, idx_v, rows_v, ob0, ob1, sem, gsem, osem):
    wid = _wid()
    iota = lax.iota(jnp.int32, 16)
    obufs = (ob0, ob1)
    pbase0 = wid * (BPW * 128)
    gv = lax.shift_right_logical(iota, 3)
    cv = lax.bitwise_and(iota, 7)

    def start_xyz(g, p):
        for h, v in ((x_hbm, xv), (y_hbm, yv), (z_hbm, zv)):
            pltpu.async_copy(h.at[pl.ds(pbase0 + g * (GROUP * 128),
                                        GROUP * 128)], v.at[p], sem)

    def wait_xyz(p):
        for h, v in ((x_hbm, xv), (y_hbm, yv), (z_hbm, zv)):
            pltpu.make_async_copy(h.at[pl.ds(0, GROUP * 128)], v.at[p], sem).wait()

    def drain_out(ob):
        for k in range(GROUP):
            for g in range(2):
                pltpu.make_async_copy(
                    ob.at[k, g, :, pl.ds(0, 128)], out_hbm.at[0, 0], osem).wait()

    start_xyz(0, 0)

    def halfiter(i, grp, p):
        start_xyz(jnp.minimum(grp + 1, NGRP - 1), 1 - p)
        wait_xyz(p)

        @plsc.parallel_loop(0, GROUP * 128, step=16, unroll=2)
        def _(s):
            sl = pl.ds(s, 16)
            h = (_quant(xv[p, sl]) ^ (_quant(yv[p, sl]) * jnp.uint32(P1))
                 ^ (_quant(zv[p, sl]) * jnp.uint32(P2)))
            idx_v[lax.shift_right_logical(s, 7),
                  pl.ds(lax.bitwise_and(s, 127), 16)] = (
                h & jnp.uint32(TABLE_MASK)).astype(jnp.int32)

        for k in range(GROUP):
            pltpu.async_copy(rm_hbm.at[idx_v.at[k]],
                             rows_v.at[pl.ds(k * 128, 128)], gsem)

        @pl.when(i >= 1)
        def _():
            drain_out(obufs[p])

        ob = obufs[p]
        bbase = wid * BPW + grp * GROUP
        for k in range(GROUP):
            pltpu.make_async_copy(rm_hbm.at[idx_v.at[k]],
                                  rows_v.at[pl.ds(k * 128, 128)], gsem).wait()
            kvec = iota * 0 + k

            @plsc.parallel_loop(0, 128, unroll=4)
            def _(p0):
                vals = rows_v[k * 128 + p0, :]
                plsc.store_scatter(ob, [kvec, gv, cv, iota * 0 + p0], vals)

            for g in range(2):
                pltpu.async_copy(ob.at[k, g, :, pl.ds(0, 128)],
                                 out_hbm.at[g, bbase + k], osem)

    def body(i, carry):
        halfiter(i, 2 * i, 0)
        halfiter(i, 2 * i + 1, 1)
        return carry

    lax.fori_loop(0, NGRP // 2, body, 0)
    drain_out(obufs[0])
    drain_out(obufs[1])
    wait_xyz(0)


_mesh = plsc.VectorSubcoreMesh(
    core_axis_name="c", subcore_axis_name="s", num_cores=NC, num_subcores=NS
)
_params = pltpu.CompilerParams(use_tc_tiling_on_sc=False,
                               needs_layout_passes=False)

_detile = pl.kernel(
    _detile_body,
    out_type=jax.ShapeDtypeStruct((N_POINTS, D), jnp.float32),
    mesh=_mesh,
    scratch_types=[
        pltpu.VMEM((2, 2, 8, 128), jnp.float32),
        pltpu.VMEM((2, 2, 8, 128), jnp.float32),
        pltpu.VMEM((256, D), jnp.float32),
        pltpu.VMEM((256, D), jnp.float32),
        pltpu.SemaphoreType.DMA,
        pltpu.SemaphoreType.DMA,
    ],
    compiler_params=_params,
)

_lookup = pl.kernel(
    _lookup_body,
    out_type=jax.ShapeDtypeStruct((2, NBLK, 8, 128), jnp.float32),
    mesh=_mesh,
    scratch_types=[
        pltpu.VMEM((2, GROUP * 128), jnp.float32),
        pltpu.VMEM((2, GROUP * 128), jnp.float32),
        pltpu.VMEM((2, GROUP * 128), jnp.float32),
        pltpu.VMEM((GROUP, 128), jnp.int32),
        pltpu.VMEM((GROUP * 128, D), jnp.float32),
        pltpu.VMEM((GROUP, 2, 8, 129), jnp.float32),
        pltpu.VMEM((GROUP, 2, 8, 129), jnp.float32),
        pltpu.SemaphoreType.DMA,
        pltpu.SemaphoreType.DMA,
        pltpu.SemaphoreType.DMA,
    ],
    compiler_params=_params,
)


@jax.jit
def kernel(xyz, table):
    table_a = table.reshape(NBLK, 128, 2, 8).transpose(2, 0, 3, 1)
    table_rm = _detile(table_a)
    out_a = _lookup(xyz[:, 0], xyz[:, 1], xyz[:, 2], table_rm)
    return out_a.transpose(1, 3, 0, 2).reshape(N_POINTS, D)


# speedup vs baseline: 1.3877x; 1.0326x over previous
import jax
import jax.numpy as jnp
from jax import lax
from jax.experimental import pallas as pl
from jax.experimental.pallas import tpu as pltpu
from jax.experimental.pallas import tpu_sc as plsc

N_POINTS = 1048576
D = 16
NBLK = N_POINTS // 128
TABLE_MASK = (1 << 20) - 1
RESOLUTION = 512.0
NC, NS = 2, 16
NW = NC * NS
BPW = NBLK // NW
NSB = BPW // 2
GROUP = 8
NGRP = BPW // GROUP

P1 = 2654435761
P2 = 805459861


def _wid():
    return lax.axis_index("s") * NC + lax.axis_index("c")


def _detile_body(ta_hbm, rm_hbm, b0, b1, rm0, rm1, sem, osem):
    wid = _wid()
    iota = lax.iota(jnp.int32, 16)
    bufs = (b0, b1)
    rms = (rm0, rm1)
    base = wid * NSB

    def start_in(sb, buf):
        for g in range(2):
            pltpu.async_copy(
                ta_hbm.at[g, pl.ds(2 * (base + sb), 2)], buf.at[g], sem)

    def wait_in(buf):
        for g in range(2):
            pltpu.make_async_copy(ta_hbm.at[0, pl.ds(0, 2)], buf.at[g], sem).wait()

    def wait_out(rm_v):
        pltpu.make_async_copy(rm_v, rm_hbm.at[pl.ds(0, 256)], osem).wait()

    start_in(0, bufs[0])

    def halfiter(j, sb, p):
        start_in(jnp.minimum(sb + 1, NSB - 1), bufs[1 - p])
        wait_in(bufs[p])

        @pl.when(j >= 1)
        def _():
            wait_out(rms[p])

        buf = bufs[p]
        rm_v = rms[p]

        @plsc.parallel_loop(0, 256, unroll=4)
        def _(q):
            blk = lax.shift_right_logical(q, 7)
            g = lax.bitwise_and(lax.shift_right_logical(q, 6), 1)
            c = lax.bitwise_and(lax.shift_right_logical(q, 3), 7)
            p0 = lax.bitwise_and(q, 7)
            vals = buf[g, blk, c, pl.ds(p0 * 16, 16)]
            rowv = iota + (blk * 128 + p0 * 16)
            colv = lax.bitwise_xor(iota, g * 8 + c)
            plsc.store_scatter(rm_v, [rowv, colv], vals)

        pltpu.async_copy(
            rm_v, rm_hbm.at[pl.ds((base + sb) * 256, 256)], osem)

    def body(j, carry):
        halfiter(j, 2 * j, 0)
        halfiter(j, 2 * j + 1, 1)
        return carry

    lax.fori_loop(0, NSB // 2, body, 0)
    wait_out(rms[0])
    wait_out(rms[1])
    wait_in(bufs[0])


def _quant(v):
    t = (v + 0.5) * RESOLUTION
    t = jnp.minimum(jnp.maximum(t, 0.0), RESOLUTION - 1.0)
    return t.astype(jnp.int32).astype(jnp.uint32)


def _lookup_body(x_hbm, y_hbm, z_hbm, rm_hbm, out_hbm,
                 xv, yv, zv, idx_v, rows_v, ob0, ob1, sem, gsem, osem):
    wid = _wid()
    iota = lax.iota(jnp.int32, 16)
    obufs = (ob0, ob1)
    pbase0 = wid * (BPW * 128)
    gv = lax.shift_right_logical(iota, 3)
    cv = lax.bitwise_and(iota, 7)

    def start_xyz(g, p):
        for h, v in ((x_hbm, xv), (y_hbm, yv), (z_hbm, zv)):
            pltpu.async_copy(h.at[pl.ds(pbase0 + g * (GROUP * 128),
                                        GROUP * 128)], v.at[p], sem)

    def wait_xyz(p):
        for h, v in ((x_hbm, xv), (y_hbm, yv), (z_hbm, zv)):
            pltpu.make_async_copy(h.at[pl.ds(0, GROUP * 128)], v.at[p], sem).wait()

    def drain_out(ob):
        for k in range(GROUP):
            for g in range(2):
                pltpu.make_async_copy(
                    ob.at[k, g, :, pl.ds(0, 128)], out_hbm.at[0, 0], osem).wait()

    start_xyz(0, 0)

    def halfiter(i, grp, p):
        start_xyz(jnp.minimum(grp + 1, NGRP - 1), 1 - p)
        wait_xyz(p)

        @plsc.parallel_loop(0, GROUP * 128, step=16, unroll=2)
        def _(s):
            sl = pl.ds(s, 16)
            h = (_quant(xv[p, sl]) ^ (_quant(yv[p, sl]) * jnp.uint32(P1))
                 ^ (_quant(zv[p, sl]) * jnp.uint32(P2)))
            idx_v[lax.shift_right_logical(s, 7),
                  pl.ds(lax.bitwise_and(s, 127), 16)] = (
                h & jnp.uint32(TABLE_MASK)).astype(jnp.int32)

        for k in range(GROUP):
            pltpu.async_copy(rm_hbm.at[idx_v.at[k]],
                             rows_v.at[pl.ds(k * 128, 128)], gsem)

        @pl.when(i >= 1)
        def _():
            drain_out(obufs[p])

        ob = obufs[p]
        bbase = wid * BPW + grp * GROUP
        for k in range(GROUP):
            pltpu.make_async_copy(rm_hbm.at[idx_v.at[k]],
                                  rows_v.at[pl.ds(k * 128, 128)], gsem).wait()
            kvec = iota * 0 + k

            @plsc.parallel_loop(0, 128, unroll=4)
            def _(p0):
                vals = rows_v[k * 128 + p0, :]
                sp = plsc.load_gather(idx_v, [kvec, iota * 0 + p0])
                cfull = lax.bitwise_xor(iota, lax.bitwise_and(sp, 15))
                plsc.store_scatter(
                    ob,
                    [kvec, lax.shift_right_logical(cfull, 3),
                     lax.bitwise_and(cfull, 7), iota * 0 + p0],
                    vals)

            for g in range(2):
                pltpu.async_copy(ob.at[k, g, :, pl.ds(0, 128)],
                                 out_hbm.at[g, bbase + k], osem)

    def body(i, carry):
        halfiter(i, 2 * i, 0)
        halfiter(i, 2 * i + 1, 1)
        return carry

    lax.fori_loop(0, NGRP // 2, body, 0)
    drain_out(obufs[0])
    drain_out(obufs[1])
    wait_xyz(0)


_mesh = plsc.VectorSubcoreMesh(
    core_axis_name="c", subcore_axis_name="s", num_cores=NC, num_subcores=NS
)
_params = pltpu.CompilerParams(use_tc_tiling_on_sc=False,
                               needs_layout_passes=False)

_detile = pl.kernel(
    _detile_body,
    out_type=jax.ShapeDtypeStruct((N_POINTS, D), jnp.float32),
    mesh=_mesh,
    scratch_types=[
        pltpu.VMEM((2, 2, 8, 128), jnp.float32),
        pltpu.VMEM((2, 2, 8, 128), jnp.float32),
        pltpu.VMEM((256, D), jnp.float32),
        pltpu.VMEM((256, D), jnp.float32),
        pltpu.SemaphoreType.DMA,
        pltpu.SemaphoreType.DMA,
    ],
    compiler_params=_params,
)

_lookup = pl.kernel(
    _lookup_body,
    out_type=jax.ShapeDtypeStruct((2, NBLK, 8, 128), jnp.float32),
    mesh=_mesh,
    scratch_types=[
        pltpu.VMEM((2, GROUP * 128), jnp.float32),
        pltpu.VMEM((2, GROUP * 128), jnp.float32),
        pltpu.VMEM((2, GROUP * 128), jnp.float32),
        pltpu.VMEM((GROUP, 128), jnp.int32),
        pltpu.VMEM((GROUP * 128, D), jnp.float32),
        pltpu.VMEM((GROUP, 2, 8, 129), jnp.float32),
        pltpu.VMEM((GROUP, 2, 8, 129), jnp.float32),
        pltpu.SemaphoreType.DMA,
        pltpu.SemaphoreType.DMA,
        pltpu.SemaphoreType.DMA,
    ],
    compiler_params=_params,
)


@jax.jit
def kernel(xyz, table):
    table_a = table.reshape(NBLK, 128, 2, 8).transpose(2, 0, 3, 1)
    table_rm = _detile(table_a)
    out_a = _lookup(xyz[:, 0], xyz[:, 1], xyz[:, 2], table_rm)
    return out_a.transpose(1, 3, 0, 2).reshape(N_POINTS, D)
